# baseline (speedup 1.0000x reference)
import functools

import jax
import jax.numpy as jnp
from jax import lax
from jax.experimental import pallas as pl
from jax.experimental.pallas import tpu as pltpu
from jax.experimental.pallas import tpu_sc as plsc

N_NODES = 10000
N_EDGES = 320000
D = 128
N_REL = 16

NC = 2
NS = 16
NW = NC * NS
E_PER_W = N_EDGES // NW
CHUNK = 128
N_FULL = E_PER_W // CHUNK
TAIL = E_PER_W - N_FULL * CHUNK
ROWS_PER_TILE = N_NODES // NS

_MESH = plsc.VectorSubcoreMesh(
    core_axis_name="c", subcore_axis_name="s", num_cores=NC, num_subcores=NS)


ZBLK = 16
N_ZBLKS = N_NODES // ZBLK
ZITERS = -(-N_ZBLKS // NS)

N_PAD_ROWS = 8
N_PAD = N_NODES + N_PAD_ROWS
CH_PER_TILE = 81
E_PAD = NW * CH_PER_TILE * CHUNK
NSLOT = 3
N_ROUNDS = CH_PER_TILE // NSLOT


def _seg_sum_body(x_hbm, src_hbm, dst_hbm, zeros_hbm, out_hbm,
                  si0, si1, si2, di0, di1, di2, rb0, rb1, rb2, agg_sh,
                  is0, is1, is2, gs0, gs1, gs2, ss0, ss1, ss2):
    c = lax.axis_index("c")
    s = lax.axis_index("s")
    w = c * NS + s
    sidx = [si0, si1, si2]
    didx = [di0, di1, di2]
    isem = [is0, is1, is2]
    rb = [rb0, rb1, rb2]
    gsem = [gs0, gs1, gs2]
    ssem = [ss0, ss1, ss2]

    def off(k):
        return (w + NW * k) * CHUNK

    def fire_idx(k, b):
        pltpu.async_copy(src_hbm.at[pl.ds(off(k), CHUNK)], sidx[b], isem[b])
        pltpu.async_copy(dst_hbm.at[pl.ds(off(k), CHUNK)], didx[b], isem[b])

    def wait_idx(b):
        pltpu.make_async_copy(src_hbm.at[pl.ds(0, CHUNK)], sidx[b], isem[b]).wait()
        pltpu.make_async_copy(dst_hbm.at[pl.ds(0, CHUNK)], didx[b], isem[b]).wait()

    def fire_g(b):
        pltpu.async_copy(x_hbm.at[sidx[b]], rb[b], gsem[b])

    def wait_g(b):
        pltpu.make_async_copy(x_hbm.at[sidx[b]], rb[b], gsem[b]).wait()

    def fire_s(b):
        pltpu.async_copy(rb[b], agg_sh.at[didx[b]], ssem[b], add=True)

    def wait_s(b):
        pltpu.make_async_copy(rb[b], agg_sh.at[didx[b]], ssem[b]).wait()

    fire_idx(0, 0)

    def zbody(k, carry):
        j = s + k * NS

        @pl.when(j < N_ZBLKS)
        def _():
            pltpu.sync_copy(zeros_hbm, agg_sh.at[pl.ds(j * ZBLK, ZBLK), :])
        return carry

    lax.fori_loop(0, ZITERS, zbody, 0)
    plsc.subcore_barrier()

    def round_body(m, carry):
        for b in range(NSLOT):
            k = m * NSLOT + b
            wait_idx(b)
            fire_g(b)

            @pl.when(k >= 1)
            def _():
                wait_g((b + 2) % NSLOT)
                fire_s((b + 2) % NSLOT)

            @pl.when(k >= 2)
            def _():
                wait_s((b + 1) % NSLOT)

            @pl.when(k < CH_PER_TILE - 1)
            def _(k=k, b=b):
                fire_idx(k + 1, (b + 1) % NSLOT)
        return carry

    lax.fori_loop(0, N_ROUNDS, round_body, 0)

    wait_g(2)
    fire_s(2)
    wait_s(1)
    wait_s(2)

    plsc.subcore_barrier()

    def obody(k, carry):
        j = s + k * NS

        @pl.when(j < N_ZBLKS)
        def _():
            pltpu.sync_copy(agg_sh.at[pl.ds(j * ZBLK, ZBLK), :],
                            out_hbm.at[c, pl.ds(j * ZBLK, ZBLK), :])
        return carry

    lax.fori_loop(0, ZITERS, obody, 0)


_seg_sum = pl.kernel(
    _seg_sum_body,
    out_type=jax.ShapeDtypeStruct((NC, N_NODES, D), jnp.float32),
    mesh=_MESH,
    scratch_types=[
        pltpu.VMEM((CHUNK,), jnp.int32),
        pltpu.VMEM((CHUNK,), jnp.int32),
        pltpu.VMEM((CHUNK,), jnp.int32),
        pltpu.VMEM((CHUNK,), jnp.int32),
        pltpu.VMEM((CHUNK,), jnp.int32),
        pltpu.VMEM((CHUNK,), jnp.int32),
        pltpu.VMEM((CHUNK, D), jnp.float32),
        pltpu.VMEM((CHUNK, D), jnp.float32),
        pltpu.VMEM((CHUNK, D), jnp.float32),
        pltpu.VMEM_SHARED((N_PAD, D), jnp.float32),
        pltpu.SemaphoreType.DMA,
        pltpu.SemaphoreType.DMA,
        pltpu.SemaphoreType.DMA,
        pltpu.SemaphoreType.DMA,
        pltpu.SemaphoreType.DMA,
        pltpu.SemaphoreType.DMA,
        pltpu.SemaphoreType.DMA,
        pltpu.SemaphoreType.DMA,
        pltpu.SemaphoreType.DMA,
    ],
)


CH_E = 2048
CH_G = CH_E // 16
N_CH = N_EDGES // CH_E
E_TAIL = N_EDGES - N_CH * CH_E
K_PER_SC = N_CH // NC
IDX_STRIPE = N_EDGES // NS


def _edge_body(u_hbm, v_hbm, src_hbm, dst_hbm, out_hbm,
               ucol, vcol, is0, is1, is2, id0, id1, id2, ob0, ob1,
               ise0, ise1, ise2, ose0, ose1):
    c = lax.axis_index("c")
    s = lax.axis_index("s")
    idx_s = [is0, is1, is2]
    idx_d = [id0, id1, id2]
    ob = [ob0, ob1]
    isem = [ise0, ise1, ise2]
    osem = [ose0, ose1]

    def jof(k):
        return c + NC * k

    def fire_idx(k, b):
        base = jof(k) * CH_E
        pltpu.async_copy(src_hbm.at[pl.ds(base, CH_E)], idx_s[b], isem[b])
        pltpu.async_copy(dst_hbm.at[pl.ds(base, CH_E)], idx_d[b], isem[b])

    def wait_idx(b):
        pltpu.make_async_copy(src_hbm.at[pl.ds(0, CH_E)], idx_s[b], isem[b]).wait()
        pltpu.make_async_copy(dst_hbm.at[pl.ds(0, CH_E)], idx_d[b], isem[b]).wait()

    def fire_out(k, b):
        pltpu.async_copy(ob[b], out_hbm.at[s, pl.ds(jof(k) * 16, 16), :], osem[b])

    def wait_out(k, b):
        pltpu.make_async_copy(ob[b], out_hbm.at[s, pl.ds(jof(k) * 16, 16), :],
                              osem[b]).wait()

    def compute(ib, b, n_groups):
        for g in range(n_groups):
            s16 = idx_s[ib][pl.ds(g * 16, 16)]
            d16 = idx_d[ib][pl.ds(g * 16, 16)]
            u16 = plsc.load_gather(ucol, [s16])
            v16 = plsc.load_gather(vcol, [d16])
            ob[b][g // 8, pl.ds((g % 8) * 16, 16)] = u16 + v16

    fire_idx(0, 0)
    fire_idx(1, 1)
    pltpu.sync_copy(u_hbm.at[s, 0, :], ucol)
    pltpu.sync_copy(v_hbm.at[s, 0, :], vcol)

    def body(m, carry):
        for r in range(6):
            k = 6 * m + r
            b = r % 2
            ib = r % 3
            wait_idx(ib)

            @pl.when(k < K_PER_SC - 2)
            def _(k=k, ib=ib):
                fire_idx(k + 2, (ib + 2) % 3)

            @pl.when(k >= 2)
            def _(k=k, b=b):
                wait_out(k - 2, b)

            compute(ib, b, CH_G)
            fire_out(k, b)
        return carry

    lax.fori_loop(0, K_PER_SC // 6, body, 0)
    wait_out(K_PER_SC - 2, 0)
    wait_out(K_PER_SC - 1, 1)

    @pl.when(c == 0)
    def _():
        base = N_CH * CH_E
        pltpu.sync_copy(src_hbm.at[pl.ds(base, E_TAIL)], is0.at[pl.ds(0, E_TAIL)])
        pltpu.sync_copy(dst_hbm.at[pl.ds(base, E_TAIL)], id0.at[pl.ds(0, E_TAIL)])
        compute(0, 0, E_TAIL // 16)
        pltpu.sync_copy(ob0.at[pl.ds(0, E_TAIL // 128), :],
                        out_hbm.at[s, pl.ds(N_CH * 16, E_TAIL // 128), :])


_edge_logits = pl.kernel(
    _edge_body,
    out_type=jax.ShapeDtypeStruct((N_REL, N_EDGES // 128, 128), jnp.float32),
    mesh=_MESH,
    scratch_types=[
        pltpu.VMEM((N_NODES,), jnp.float32),
        pltpu.VMEM((N_NODES,), jnp.float32),
        pltpu.VMEM((CH_E,), jnp.int32),
        pltpu.VMEM((CH_E,), jnp.int32),
        pltpu.VMEM((CH_E,), jnp.int32),
        pltpu.VMEM((CH_E,), jnp.int32),
        pltpu.VMEM((CH_E,), jnp.int32),
        pltpu.VMEM((CH_E,), jnp.int32),
        pltpu.VMEM((16, 128), jnp.float32),
        pltpu.VMEM((16, 128), jnp.float32),
        pltpu.SemaphoreType.DMA,
        pltpu.SemaphoreType.DMA,
        pltpu.SemaphoreType.DMA,
        pltpu.SemaphoreType.DMA,
        pltpu.SemaphoreType.DMA,
    ],
    compiler_params=pltpu.CompilerParams(needs_layout_passes=False),
)


ROW_BLK = 2000
N_BLKS = N_NODES // ROW_BLK


def _lin1_body(a0, a1, x, wrel, wroot, b, o):
    agg = a0[...] + a1[...]
    h = jnp.dot(agg, wrel[...], preferred_element_type=jnp.float32)
    h = h + jnp.dot(x[...], wroot[...], preferred_element_type=jnp.float32)
    o[...] = jnp.maximum(h + b[...], 0.0)


def _lin2_body(a0, a1, x, wrel, wroot, b, wet, web, be_r, u, v, m):
    agg = a0[...] + a1[...]
    h = jnp.dot(agg, wrel[...], preferred_element_type=jnp.float32)
    h = h + jnp.dot(x[...], wroot[...], preferred_element_type=jnp.float32)
    x2 = jnp.maximum(h + b[...], 0.0)
    u[...] = jnp.dot(x2, wet[...], preferred_element_type=jnp.float32) + be_r[...]
    v[...] = jnp.dot(x2, web[...], preferred_element_type=jnp.float32)

    @pl.when(pl.program_id(0) == 0)
    def _():
        m[...] = jnp.zeros_like(m)

    m[...] += jnp.sum(x2, axis=0, keepdims=True) * (1.0 / N_NODES)


_row_spec = pl.BlockSpec((ROW_BLK, D), lambda i: (i, 0))
_w_spec = pl.BlockSpec((D, D), lambda i: (0, 0))
_b_spec = pl.BlockSpec((1, D), lambda i: (0, 0))

_lin1 = pl.pallas_call(
    _lin1_body,
    grid=(N_BLKS,),
    in_specs=[_row_spec, _row_spec, _row_spec, _w_spec, _w_spec, _b_spec],
    out_specs=_row_spec,
    out_shape=jax.ShapeDtypeStruct((N_NODES, D), jnp.float32),
)

_lin2 = pl.pallas_call(
    _lin2_body,
    grid=(N_BLKS,),
    in_specs=[_row_spec, _row_spec, _row_spec, _w_spec, _w_spec, _b_spec,
              pl.BlockSpec((D, N_REL), lambda i: (0, 0)),
              pl.BlockSpec((D, N_REL), lambda i: (0, 0)),
              pl.BlockSpec((1, N_REL), lambda i: (0, 0))],
    out_specs=[pl.BlockSpec((ROW_BLK, N_REL), lambda i: (i, 0)),
               pl.BlockSpec((ROW_BLK, N_REL), lambda i: (i, 0)),
               pl.BlockSpec((1, D), lambda i: (0, 0))],
    out_shape=[jax.ShapeDtypeStruct((N_NODES, N_REL), jnp.float32),
               jax.ShapeDtypeStruct((N_NODES, N_REL), jnp.float32),
               jax.ShapeDtypeStruct((1, D), jnp.float32)],
)


def kernel(node_feats, edge_index, W1_rel, b1, W1_root, W2_rel, b2, W2_root, We, be):
    src = edge_index[0].astype(jnp.int32)
    dst = edge_index[1].astype(jnp.int32)
    zeros_blk = jnp.zeros((ZBLK, D), jnp.float32)

    pad_lanes = jnp.arange(E_PAD - N_EDGES, dtype=jnp.int32) % N_PAD_ROWS
    src_p = jnp.concatenate([src, pad_lanes])
    dst_p = jnp.concatenate([dst, N_NODES + pad_lanes])

    agg1 = _seg_sum(node_feats, src_p, dst_p, zeros_blk)
    x1 = _lin1(agg1[0], agg1[1], node_feats, W1_rel, W1_root, b1.reshape(1, D))

    agg2 = _seg_sum(x1, src_p, dst_p, zeros_blk)
    u, v, graph_embed = _lin2(agg2[0], agg2[1], x1, W2_rel, W2_root,
                              b2.reshape(1, D), We[:D], We[D:],
                              be.reshape(1, N_REL))

    u_t = u.T.reshape(N_REL, 1, N_NODES)
    v_t = v.T.reshape(N_REL, 1, N_NODES)
    out_t = _edge_logits(u_t, v_t, src, dst)
    edge_logits = out_t.reshape(N_REL, N_EDGES).T
    return (edge_logits, graph_embed)


# speedup vs baseline: 1.0436x; 1.0436x over previous
import functools

import jax
import jax.numpy as jnp
from jax import lax
from jax.experimental import pallas as pl
from jax.experimental.pallas import tpu as pltpu
from jax.experimental.pallas import tpu_sc as plsc

N_NODES = 10000
N_EDGES = 320000
D = 128
N_REL = 16

NC = 2
NS = 16
NW = NC * NS
E_PER_W = N_EDGES // NW
CHUNK = 128
N_FULL = E_PER_W // CHUNK
TAIL = E_PER_W - N_FULL * CHUNK
ROWS_PER_TILE = N_NODES // NS

_MESH = plsc.VectorSubcoreMesh(
    core_axis_name="c", subcore_axis_name="s", num_cores=NC, num_subcores=NS)


ZBLK = 16
N_ZBLKS = N_NODES // ZBLK
ZITERS = -(-N_ZBLKS // NS)

N_PAD_ROWS = 8
N_PAD = N_NODES + N_PAD_ROWS
CH_PER_TILE = 81
E_PAD = NW * CH_PER_TILE * CHUNK
NSLOT = 3
N_ROUNDS = CH_PER_TILE // NSLOT


def _seg_sum_body(x_hbm, src_hbm, dst_hbm, zeros_hbm, out_hbm,
                  si0, si1, si2, di0, di1, di2, rb0, rb1, rb2, agg_sh,
                  is0, is1, is2, gs0, gs1, gs2, ss0, ss1, ss2):
    c = lax.axis_index("c")
    s = lax.axis_index("s")
    w = c * NS + s
    sidx = [si0, si1, si2]
    didx = [di0, di1, di2]
    isem = [is0, is1, is2]
    rb = [rb0, rb1, rb2]
    gsem = [gs0, gs1, gs2]
    ssem = [ss0, ss1, ss2]

    def off(k):
        return (w + NW * k) * CHUNK

    def fire_idx(k, b):
        pltpu.async_copy(src_hbm.at[pl.ds(off(k), CHUNK)], sidx[b], isem[b])
        pltpu.async_copy(dst_hbm.at[pl.ds(off(k), CHUNK)], didx[b], isem[b])

    def wait_idx(b):
        pltpu.make_async_copy(src_hbm.at[pl.ds(0, CHUNK)], sidx[b], isem[b]).wait()
        pltpu.make_async_copy(dst_hbm.at[pl.ds(0, CHUNK)], didx[b], isem[b]).wait()

    def fire_g(b):
        pltpu.async_copy(x_hbm.at[sidx[b]], rb[b], gsem[b])

    def wait_g(b):
        pltpu.make_async_copy(x_hbm.at[sidx[b]], rb[b], gsem[b]).wait()

    def fire_s(b):
        pltpu.async_copy(rb[b], agg_sh.at[didx[b]], ssem[b], add=True)

    def wait_s(b):
        pltpu.make_async_copy(rb[b], agg_sh.at[didx[b]], ssem[b]).wait()

    fire_idx(0, 0)

    def zbody(k, carry):
        j = s + k * NS

        @pl.when(j < N_ZBLKS)
        def _():
            pltpu.sync_copy(zeros_hbm, agg_sh.at[pl.ds(j * ZBLK, ZBLK), :])
        return carry

    lax.fori_loop(0, ZITERS, zbody, 0)
    plsc.subcore_barrier()

    def round_body(m, carry):
        for b in range(NSLOT):
            k = m * NSLOT + b
            wait_idx(b)
            fire_g(b)

            @pl.when(k >= 1)
            def _():
                wait_g((b + 2) % NSLOT)
                fire_s((b + 2) % NSLOT)

            @pl.when(k >= 2)
            def _():
                wait_s((b + 1) % NSLOT)

            @pl.when(k < CH_PER_TILE - 1)
            def _(k=k, b=b):
                fire_idx(k + 1, (b + 1) % NSLOT)
        return carry

    lax.fori_loop(0, N_ROUNDS, round_body, 0)

    wait_g(2)
    fire_s(2)
    wait_s(1)
    wait_s(2)

    plsc.subcore_barrier()

    def obody(k, carry):
        j = s + k * NS

        @pl.when(j < N_ZBLKS)
        def _():
            pltpu.sync_copy(agg_sh.at[pl.ds(j * ZBLK, ZBLK), :],
                            out_hbm.at[c, pl.ds(j * ZBLK, ZBLK), :])
        return carry

    lax.fori_loop(0, ZITERS, obody, 0)


_seg_sum = pl.kernel(
    _seg_sum_body,
    out_type=jax.ShapeDtypeStruct((NC, N_NODES, D), jnp.float32),
    mesh=_MESH,
    scratch_types=[
        pltpu.VMEM((CHUNK,), jnp.int32),
        pltpu.VMEM((CHUNK,), jnp.int32),
        pltpu.VMEM((CHUNK,), jnp.int32),
        pltpu.VMEM((CHUNK,), jnp.int32),
        pltpu.VMEM((CHUNK,), jnp.int32),
        pltpu.VMEM((CHUNK,), jnp.int32),
        pltpu.VMEM((CHUNK, D), jnp.float32),
        pltpu.VMEM((CHUNK, D), jnp.float32),
        pltpu.VMEM((CHUNK, D), jnp.float32),
        pltpu.VMEM_SHARED((N_PAD, D), jnp.float32),
        pltpu.SemaphoreType.DMA,
        pltpu.SemaphoreType.DMA,
        pltpu.SemaphoreType.DMA,
        pltpu.SemaphoreType.DMA,
        pltpu.SemaphoreType.DMA,
        pltpu.SemaphoreType.DMA,
        pltpu.SemaphoreType.DMA,
        pltpu.SemaphoreType.DMA,
        pltpu.SemaphoreType.DMA,
    ],
)


CH_E = 2048
CH_G = CH_E // 16
N_CH = N_EDGES // CH_E
E_TAIL = N_EDGES - N_CH * CH_E
K_PER_SC = N_CH // NC
IDX_STRIPE = N_EDGES // NS


def _edge_body(u_hbm, v_hbm, src_hbm, dst_hbm, out_hbm,
               ucol, vcol, is0, is1, id0, id1, ob0, ob1,
               ise0, ise1, ose0, ose1):
    c = lax.axis_index("c")
    s = lax.axis_index("s")
    idx_s = [is0, is1]
    idx_d = [id0, id1]
    ob = [ob0, ob1]
    isem = [ise0, ise1]
    osem = [ose0, ose1]

    def jof(k):
        return c + NC * k

    def fire_idx(k, b):
        base = jof(k) * CH_E
        pltpu.async_copy(src_hbm.at[pl.ds(base, CH_E)], idx_s[b], isem[b])
        pltpu.async_copy(dst_hbm.at[pl.ds(base, CH_E)], idx_d[b], isem[b])

    def wait_idx(b):
        pltpu.make_async_copy(src_hbm.at[pl.ds(0, CH_E)], idx_s[b], isem[b]).wait()
        pltpu.make_async_copy(dst_hbm.at[pl.ds(0, CH_E)], idx_d[b], isem[b]).wait()

    def fire_out(k, b):
        pltpu.async_copy(ob[b], out_hbm.at[s, pl.ds(jof(k) * 16, 16), :], osem[b])

    def wait_out(k, b):
        pltpu.make_async_copy(ob[b], out_hbm.at[s, pl.ds(jof(k) * 16, 16), :],
                              osem[b]).wait()

    def compute(b, n_groups):
        for g in range(n_groups):
            s16 = idx_s[b][pl.ds(g * 16, 16)]
            d16 = idx_d[b][pl.ds(g * 16, 16)]
            u16 = plsc.load_gather(ucol, [s16])
            v16 = plsc.load_gather(vcol, [d16])
            ob[b][g // 8, pl.ds((g % 8) * 16, 16)] = u16 + v16

    fire_idx(0, 0)
    pltpu.sync_copy(u_hbm.at[s, 0, :], ucol)
    pltpu.sync_copy(v_hbm.at[s, 0, :], vcol)

    def body(m, carry):
        for b in range(2):
            k = 2 * m + b
            wait_idx(b)

            @pl.when(k < K_PER_SC - 1)
            def _(k=k, b=b):
                fire_idx(k + 1, 1 - b)

            @pl.when(k >= 2)
            def _(k=k, b=b):
                wait_out(k - 2, b)

            compute(b, CH_G)
            fire_out(k, b)
        return carry

    lax.fori_loop(0, K_PER_SC // 2, body, 0)
    wait_out(K_PER_SC - 2, 0)
    wait_out(K_PER_SC - 1, 1)

    @pl.when(c == 0)
    def _():
        base = N_CH * CH_E
        pltpu.sync_copy(src_hbm.at[pl.ds(base, E_TAIL)], is0.at[pl.ds(0, E_TAIL)])
        pltpu.sync_copy(dst_hbm.at[pl.ds(base, E_TAIL)], id0.at[pl.ds(0, E_TAIL)])
        compute(0, E_TAIL // 16)
        pltpu.sync_copy(ob0.at[pl.ds(0, E_TAIL // 128), :],
                        out_hbm.at[s, pl.ds(N_CH * 16, E_TAIL // 128), :])


_edge_logits = pl.kernel(
    _edge_body,
    out_type=jax.ShapeDtypeStruct((N_REL, N_EDGES // 128, 128), jnp.float32),
    mesh=_MESH,
    scratch_types=[
        pltpu.VMEM((N_NODES,), jnp.float32),
        pltpu.VMEM((N_NODES,), jnp.float32),
        pltpu.VMEM((CH_E,), jnp.int32),
        pltpu.VMEM((CH_E,), jnp.int32),
        pltpu.VMEM((CH_E,), jnp.int32),
        pltpu.VMEM((CH_E,), jnp.int32),
        pltpu.VMEM((16, 128), jnp.float32),
        pltpu.VMEM((16, 128), jnp.float32),
        pltpu.SemaphoreType.DMA,
        pltpu.SemaphoreType.DMA,
        pltpu.SemaphoreType.DMA,
        pltpu.SemaphoreType.DMA,
    ],
    compiler_params=pltpu.CompilerParams(needs_layout_passes=False),
)


ROW_BLK = 2000
N_BLKS = N_NODES // ROW_BLK


def _lin1_body(a0, a1, x, wrel, wroot, b, o):
    agg = a0[...] + a1[...]
    h = jnp.dot(agg, wrel[...], preferred_element_type=jnp.float32)
    h = h + jnp.dot(x[...], wroot[...], preferred_element_type=jnp.float32)
    o[...] = jnp.maximum(h + b[...], 0.0)


def _lin2_body(a0, a1, x, wrel, wroot, b, wet, web, be_r, u, v, m):
    agg = a0[...] + a1[...]
    h = jnp.dot(agg, wrel[...], preferred_element_type=jnp.float32)
    h = h + jnp.dot(x[...], wroot[...], preferred_element_type=jnp.float32)
    x2 = jnp.maximum(h + b[...], 0.0)
    u[...] = jnp.dot(x2, wet[...], preferred_element_type=jnp.float32) + be_r[...]
    v[...] = jnp.dot(x2, web[...], preferred_element_type=jnp.float32)

    @pl.when(pl.program_id(0) == 0)
    def _():
        m[...] = jnp.zeros_like(m)

    m[...] += jnp.sum(x2, axis=0, keepdims=True) * (1.0 / N_NODES)


_row_spec = pl.BlockSpec((ROW_BLK, D), lambda i: (i, 0))
_w_spec = pl.BlockSpec((D, D), lambda i: (0, 0))
_b_spec = pl.BlockSpec((1, D), lambda i: (0, 0))

_lin1 = pl.pallas_call(
    _lin1_body,
    grid=(N_BLKS,),
    in_specs=[_row_spec, _row_spec, _row_spec, _w_spec, _w_spec, _b_spec],
    out_specs=_row_spec,
    out_shape=jax.ShapeDtypeStruct((N_NODES, D), jnp.float32),
)

_lin2 = pl.pallas_call(
    _lin2_body,
    grid=(N_BLKS,),
    in_specs=[_row_spec, _row_spec, _row_spec, _w_spec, _w_spec, _b_spec,
              pl.BlockSpec((D, N_REL), lambda i: (0, 0)),
              pl.BlockSpec((D, N_REL), lambda i: (0, 0)),
              pl.BlockSpec((1, N_REL), lambda i: (0, 0))],
    out_specs=[pl.BlockSpec((ROW_BLK, N_REL), lambda i: (i, 0)),
               pl.BlockSpec((ROW_BLK, N_REL), lambda i: (i, 0)),
               pl.BlockSpec((1, D), lambda i: (0, 0))],
    out_shape=[jax.ShapeDtypeStruct((N_NODES, N_REL), jnp.float32),
               jax.ShapeDtypeStruct((N_NODES, N_REL), jnp.float32),
               jax.ShapeDtypeStruct((1, D), jnp.float32)],
)


def kernel(node_feats, edge_index, W1_rel, b1, W1_root, W2_rel, b2, W2_root, We, be):
    src = edge_index[0].astype(jnp.int32)
    dst = edge_index[1].astype(jnp.int32)
    zeros_blk = jnp.zeros((ZBLK, D), jnp.float32)

    pad_lanes = jnp.arange(E_PAD - N_EDGES, dtype=jnp.int32) % N_PAD_ROWS
    src_p = jnp.concatenate([src, pad_lanes])
    dst_p = jnp.concatenate([dst, N_NODES + pad_lanes])

    agg1 = _seg_sum(node_feats, src_p, dst_p, zeros_blk)
    x1 = _lin1(agg1[0], agg1[1], node_feats, W1_rel, W1_root, b1.reshape(1, D))

    agg2 = _seg_sum(x1, src_p, dst_p, zeros_blk)
    u, v, graph_embed = _lin2(agg2[0], agg2[1], x1, W2_rel, W2_root,
                              b2.reshape(1, D), We[:D], We[D:],
                              be.reshape(1, N_REL))

    u_t = u.T.reshape(N_REL, 1, N_NODES)
    v_t = v.T.reshape(N_REL, 1, N_NODES)
    out_t = _edge_logits(u_t, v_t, src, dst)
    edge_logits = out_t.reshape(N_REL, N_EDGES).T
    return (edge_logits, graph_embed)
